# Initial kernel scaffold; baseline (speedup 1.0000x reference)
import jax
import jax.numpy as jnp
from jax.experimental import pallas as pl


def kernel(pred_boxes, scores):
    raise NotImplementedError("write your pallas kernel here")


# speedup vs baseline: 12.4898x; 12.4898x over previous
import functools

import jax
import jax.numpy as jnp
from jax.experimental import pallas as pl
from jax.experimental.pallas import tpu as pltpu

_THRESH = 0.3
_B = 512


def _nms_kernel(vals_ref, valsT_ref, out_ref, srtT_ref, rankc_ref, rankr_ref,
                keep_ref, sup_ref, *, nb, B):
    f32 = jnp.float32
    i32 = jnp.int32
    lane_b = jax.lax.broadcasted_iota(i32, (1, B), 1)
    sub_b = jax.lax.broadcasted_iota(i32, (B, 1), 0)
    hi = jax.lax.Precision.HIGHEST

    for r in range(nb):
        b0 = r * B
        s_i = valsT_ref[b0:b0 + B, 4:5]
        i_ids = sub_b + b0
        s_i2 = vals_ref[4:5, b0:b0 + B]
        i_ids2 = lane_b + b0

        def rank_body(c, carry):
            cnt_c, cnt_r = carry
            c0 = pl.multiple_of(c * B, B)
            s_j = vals_ref[4:5, pl.ds(c0, B)]
            j_ids = lane_b + c * B
            beats = (s_j > s_i) | ((s_j == s_i) & (j_ids < i_ids))
            cnt_c = cnt_c + jnp.sum(beats.astype(i32), axis=1, keepdims=True)
            s_j2 = valsT_ref[pl.ds(c0, B), 4:5]
            j_ids2 = sub_b + c * B
            beats2 = (s_j2 > s_i2) | ((s_j2 == s_i2) & (j_ids2 < i_ids2))
            cnt_r = cnt_r + jnp.sum(beats2.astype(i32), axis=0, keepdims=True)
            return cnt_c, cnt_r

        cnt_c, cnt_r = jax.lax.fori_loop(
            0, nb, rank_body, (jnp.zeros((B, 1), i32), jnp.zeros((1, B), i32)))
        rankc_ref[b0:b0 + B, :] = cnt_c
        rankr_ref[:, b0:b0 + B] = cnt_r

    for c in range(nb):
        c0 = c * B
        dest_c = sub_b + c0
        dest_r = lane_b + c0

        def perm_body(r, carry):
            acc_c, accT_c = carry
            r0 = pl.multiple_of(r * B, B)
            rc = rankc_ref[pl.ds(r0, B), :]
            oh = (rc == dest_r).astype(f32)
            acc_c = acc_c + jnp.dot(vals_ref[:, pl.ds(r0, B)], oh,
                                    preferred_element_type=f32, precision=hi)
            rr = rankr_ref[:, pl.ds(r0, B)]
            oh2 = (dest_c == rr).astype(f32)
            accT_c = accT_c + jnp.dot(oh2, valsT_ref[pl.ds(r0, B), :],
                                      preferred_element_type=f32, precision=hi)
            return acc_c, accT_c

        acc_c, accT_c = jax.lax.fori_loop(
            0, nb, perm_body,
            (jnp.zeros((8, B), f32), jnp.zeros((B, 8), f32)))
        out_ref[:, c0:c0 + B] = acc_c
        srtT_ref[c0:c0 + B, :] = accT_c

    keep_ref[...] = jnp.ones_like(keep_ref)
    tri = (jax.lax.broadcasted_iota(i32, (B, B), 1)
           > jax.lax.broadcasted_iota(i32, (B, B), 0))

    def iou_tile(i0, j0):
        x1c = srtT_ref[i0:i0 + B, 0:1]
        y1c = srtT_ref[i0:i0 + B, 1:2]
        x2c = srtT_ref[i0:i0 + B, 2:3]
        y2c = srtT_ref[i0:i0 + B, 3:4]
        a_col = (x2c - x1c + 1.0) * (y2c - y1c + 1.0)
        x1r = out_ref[0:1, pl.ds(j0, B)]
        y1r = out_ref[1:2, pl.ds(j0, B)]
        x2r = out_ref[2:3, pl.ds(j0, B)]
        y2r = out_ref[3:4, pl.ds(j0, B)]
        a_row = (x2r - x1r + 1.0) * (y2r - y1r + 1.0)
        xx1 = jnp.maximum(x1c, x1r)
        yy1 = jnp.maximum(y1c, y1r)
        xx2 = jnp.minimum(x2c, x2r)
        yy2 = jnp.minimum(y2c, y2r)
        iw = jnp.maximum(xx2 - xx1 + 1.0, 0.0)
        ih = jnp.maximum(yy2 - yy1 + 1.0, 0.0)
        inter = iw * ih
        union = (a_col + a_row) - inter
        return inter / union

    for bi in range(nb):
        b0 = bi * B
        sup_ref[...] = ((iou_tile(b0, b0) > _THRESH) & tri).astype(f32)

        def self_body(i, kb):
            row = sup_ref[pl.ds(i, 1), :]
            ki = jnp.sum(kb * (lane_b == i).astype(f32))
            return kb * (1.0 - ki * row)

        kb = jax.lax.fori_loop(0, B, self_body, keep_ref[0:1, b0:b0 + B])
        keep_ref[0:1, b0:b0 + B] = kb

        def cross_body(rj, _):
            j0 = pl.multiple_of(rj * B, B)
            supm = (iou_tile(b0, j0) > _THRESH).astype(f32)
            cnt = jnp.dot(kb, supm, preferred_element_type=f32, precision=hi)
            keep_ref[0:1, pl.ds(j0, B)] = (keep_ref[0:1, pl.ds(j0, B)]
                                           * (cnt < 0.5).astype(f32))
            return 0

        if bi + 1 < nb:
            jax.lax.fori_loop(bi + 1, nb, cross_body, 0)

    out_ref[...] = out_ref[...] * keep_ref[...]


def kernel(pred_boxes, scores):
    N = pred_boxes.shape[0]
    B = _B
    P = ((N + B - 1) // B) * B
    pad = P - N
    boxes_p = jnp.pad(pred_boxes.astype(jnp.float32), ((0, pad), (0, 0)))
    scores_p = jnp.pad(scores.astype(jnp.float32), (0, pad),
                       constant_values=-1.0)
    vals8 = jnp.concatenate(
        [boxes_p.T, scores_p[None, :], jnp.zeros((3, P), jnp.float32)], axis=0)
    valsT = vals8.T
    out = pl.pallas_call(
        functools.partial(_nms_kernel, nb=P // B, B=B),
        out_shape=jax.ShapeDtypeStruct((8, P), jnp.float32),
        scratch_shapes=[
            pltpu.VMEM((P, 8), jnp.float32),
            pltpu.VMEM((P, 1), jnp.int32),
            pltpu.VMEM((1, P), jnp.int32),
            pltpu.VMEM((1, P), jnp.float32),
            pltpu.VMEM((B, B), jnp.float32),
        ],
    )(vals8, valsT)
    return out[:5, :N].T


# speedup vs baseline: 40.4663x; 3.2400x over previous
import functools

import jax
import jax.numpy as jnp
from jax.experimental import pallas as pl
from jax.experimental.pallas import tpu as pltpu

_THRESH = 0.3
_B = 512


def _nms_kernel(vals_ref, valsT_ref, out_ref, srtT_ref, rankc_ref, rankr_ref,
                keep_ref, sup_ref, *, nb, B):
    f32 = jnp.float32
    i32 = jnp.int32
    lane_b = jax.lax.broadcasted_iota(i32, (1, B), 1)
    sub_b = jax.lax.broadcasted_iota(i32, (B, 1), 0)
    hi = jax.lax.Precision.HIGHEST

    for r in range(nb):
        b0 = r * B
        s_i = valsT_ref[b0:b0 + B, 4:5]
        i_ids = sub_b + b0
        s_i2 = vals_ref[4:5, b0:b0 + B]
        i_ids2 = lane_b + b0

        def rank_body(c, carry):
            cnt_c, cnt_r = carry
            c0 = pl.multiple_of(c * B, B)
            s_j = vals_ref[4:5, pl.ds(c0, B)]
            j_ids = lane_b + c * B
            beats = (s_j > s_i) | ((s_j == s_i) & (j_ids < i_ids))
            cnt_c = cnt_c + jnp.sum(beats.astype(i32), axis=1, keepdims=True)
            s_j2 = valsT_ref[pl.ds(c0, B), 4:5]
            j_ids2 = sub_b + c * B
            beats2 = (s_j2 > s_i2) | ((s_j2 == s_i2) & (j_ids2 < i_ids2))
            cnt_r = cnt_r + jnp.sum(beats2.astype(i32), axis=0, keepdims=True)
            return cnt_c, cnt_r

        cnt_c, cnt_r = jax.lax.fori_loop(
            0, nb, rank_body, (jnp.zeros((B, 1), i32), jnp.zeros((1, B), i32)))
        rankc_ref[b0:b0 + B, :] = cnt_c
        rankr_ref[:, b0:b0 + B] = cnt_r

    for c in range(nb):
        c0 = c * B
        dest_c = sub_b + c0
        dest_r = lane_b + c0

        def perm_body(r, carry):
            acc_c, accT_c = carry
            r0 = pl.multiple_of(r * B, B)
            rc = rankc_ref[pl.ds(r0, B), :]
            oh = (rc == dest_r).astype(f32)
            acc_c = acc_c + jnp.dot(vals_ref[:, pl.ds(r0, B)], oh,
                                    preferred_element_type=f32, precision=hi)
            rr = rankr_ref[:, pl.ds(r0, B)]
            oh2 = (dest_c == rr).astype(f32)
            accT_c = accT_c + jnp.dot(oh2, valsT_ref[pl.ds(r0, B), :],
                                      preferred_element_type=f32, precision=hi)
            return acc_c, accT_c

        acc_c, accT_c = jax.lax.fori_loop(
            0, nb, perm_body,
            (jnp.zeros((8, B), f32), jnp.zeros((B, 8), f32)))
        out_ref[:, c0:c0 + B] = acc_c
        srtT_ref[c0:c0 + B, :] = accT_c

    keep_ref[...] = jnp.ones_like(keep_ref)
    tri = (jax.lax.broadcasted_iota(i32, (B, B), 1)
           > jax.lax.broadcasted_iota(i32, (B, B), 0))

    def iou_tile(i0, j0):
        x1c = srtT_ref[i0:i0 + B, 0:1]
        y1c = srtT_ref[i0:i0 + B, 1:2]
        x2c = srtT_ref[i0:i0 + B, 2:3]
        y2c = srtT_ref[i0:i0 + B, 3:4]
        a_col = (x2c - x1c + 1.0) * (y2c - y1c + 1.0)
        x1r = out_ref[0:1, pl.ds(j0, B)]
        y1r = out_ref[1:2, pl.ds(j0, B)]
        x2r = out_ref[2:3, pl.ds(j0, B)]
        y2r = out_ref[3:4, pl.ds(j0, B)]
        a_row = (x2r - x1r + 1.0) * (y2r - y1r + 1.0)
        xx1 = jnp.maximum(x1c, x1r)
        yy1 = jnp.maximum(y1c, y1r)
        xx2 = jnp.minimum(x2c, x2r)
        yy2 = jnp.minimum(y2c, y2r)
        iw = jnp.maximum(xx2 - xx1 + 1.0, 0.0)
        ih = jnp.maximum(yy2 - yy1 + 1.0, 0.0)
        inter = iw * ih
        union = (a_col + a_row) - inter
        return inter / union

    for bi in range(nb):
        b0 = bi * B
        sup_ref[...] = ((iou_tile(b0, b0) > _THRESH) & tri).astype(f32)
        k0 = keep_ref[0:1, b0:b0 + B]

        def self_cond(carry):
            return carry[1]

        def self_body(carry):
            k, _ = carry
            cnt = jnp.dot(k, sup_ref[...], preferred_element_type=f32)
            knew = k0 * (cnt < 0.5).astype(f32)
            return knew, jnp.any(knew != k)

        kb, _ = jax.lax.while_loop(self_cond, self_body,
                                   (k0, jnp.bool_(True)))
        keep_ref[0:1, b0:b0 + B] = kb

        def cross_body(rj, _):
            j0 = pl.multiple_of(rj * B, B)
            supm = (iou_tile(b0, j0) > _THRESH).astype(f32)
            cnt = jnp.dot(kb, supm, preferred_element_type=f32)
            keep_ref[0:1, pl.ds(j0, B)] = (keep_ref[0:1, pl.ds(j0, B)]
                                           * (cnt < 0.5).astype(f32))
            return 0

        if bi + 1 < nb:
            jax.lax.fori_loop(bi + 1, nb, cross_body, 0)

    out_ref[...] = out_ref[...] * keep_ref[...]


def kernel(pred_boxes, scores):
    N = pred_boxes.shape[0]
    B = _B
    P = ((N + B - 1) // B) * B
    pad = P - N
    boxes_p = jnp.pad(pred_boxes.astype(jnp.float32), ((0, pad), (0, 0)))
    scores_p = jnp.pad(scores.astype(jnp.float32), (0, pad),
                       constant_values=-1.0)
    vals8 = jnp.concatenate(
        [boxes_p.T, scores_p[None, :], jnp.zeros((3, P), jnp.float32)], axis=0)
    valsT = vals8.T
    out = pl.pallas_call(
        functools.partial(_nms_kernel, nb=P // B, B=B),
        out_shape=jax.ShapeDtypeStruct((8, P), jnp.float32),
        scratch_shapes=[
            pltpu.VMEM((P, 8), jnp.float32),
            pltpu.VMEM((P, 1), jnp.int32),
            pltpu.VMEM((1, P), jnp.int32),
            pltpu.VMEM((1, P), jnp.float32),
            pltpu.VMEM((B, B), jnp.float32),
        ],
    )(vals8, valsT)
    return out[:5, :N].T


# speedup vs baseline: 43.5071x; 1.0751x over previous
import functools

import jax
import jax.numpy as jnp
from jax.experimental import pallas as pl
from jax.experimental.pallas import tpu as pltpu

_THRESH = 0.3
_B = 512


def _nms_kernel(vals_ref, valsT_ref, out_ref, srtT_ref, rankc_ref,
                keep_ref, sup_ref, *, nb, B):
    f32 = jnp.float32
    i32 = jnp.int32
    lane_b = jax.lax.broadcasted_iota(i32, (1, B), 1)
    sub_b = jax.lax.broadcasted_iota(i32, (B, 1), 0)
    hi = jax.lax.Precision.HIGHEST

    ones_col = jnp.ones((B, 1), f32)
    for r in range(nb):
        b0 = r * B
        s_i = valsT_ref[b0:b0 + B, 4:5]
        i_ids = sub_b + b0

        def rank_body(c, cnt):
            c0 = pl.multiple_of(c * B, B)
            s_j = vals_ref[4:5, pl.ds(c0, B)]
            j_ids = lane_b + c * B
            beats = (s_j > s_i) | ((s_j == s_i) & (j_ids < i_ids))
            bf = jnp.where(beats, 1.0, 0.0)
            return cnt + jnp.dot(bf, ones_col, preferred_element_type=f32)

        cnt_c = jax.lax.fori_loop(0, nb, rank_body, jnp.zeros((B, 1), f32))
        rankc_ref[b0:b0 + B, :] = cnt_c

    for c in range(nb):
        c0 = c * B
        dest_r = lane_b + c0

        def perm_body(r, carry):
            acc_c, accT_c = carry
            r0 = pl.multiple_of(r * B, B)
            rc = rankc_ref[pl.ds(r0, B), :]
            oh = (rc == dest_r.astype(f32)).astype(f32)
            acc_c = acc_c + jnp.dot(vals_ref[:, pl.ds(r0, B)], oh,
                                    preferred_element_type=f32, precision=hi)
            oh2 = jnp.transpose(oh)
            accT_c = accT_c + jnp.dot(oh2, valsT_ref[pl.ds(r0, B), :],
                                      preferred_element_type=f32, precision=hi)
            return acc_c, accT_c

        acc_c, accT_c = jax.lax.fori_loop(
            0, nb, perm_body,
            (jnp.zeros((8, B), f32), jnp.zeros((B, 8), f32)))
        out_ref[:, c0:c0 + B] = acc_c
        srtT_ref[c0:c0 + B, :] = accT_c

    keep_ref[...] = jnp.ones_like(keep_ref)
    tri = (jax.lax.broadcasted_iota(i32, (B, B), 1)
           > jax.lax.broadcasted_iota(i32, (B, B), 0))

    def iou_tile(i0, j0):
        x1c = srtT_ref[i0:i0 + B, 0:1]
        y1c = srtT_ref[i0:i0 + B, 1:2]
        x2c = srtT_ref[i0:i0 + B, 2:3]
        y2c = srtT_ref[i0:i0 + B, 3:4]
        a_col = (x2c - x1c + 1.0) * (y2c - y1c + 1.0)
        x1r = out_ref[0:1, pl.ds(j0, B)]
        y1r = out_ref[1:2, pl.ds(j0, B)]
        x2r = out_ref[2:3, pl.ds(j0, B)]
        y2r = out_ref[3:4, pl.ds(j0, B)]
        a_row = (x2r - x1r + 1.0) * (y2r - y1r + 1.0)
        xx1 = jnp.maximum(x1c, x1r)
        yy1 = jnp.maximum(y1c, y1r)
        xx2 = jnp.minimum(x2c, x2r)
        yy2 = jnp.minimum(y2c, y2r)
        iw = jnp.maximum(xx2 - xx1 + 1.0, 0.0)
        ih = jnp.maximum(yy2 - yy1 + 1.0, 0.0)
        inter = iw * ih
        union = (a_col + a_row) - inter
        return inter / union

    for bi in range(nb):
        b0 = bi * B
        sup_ref[...] = ((iou_tile(b0, b0) > _THRESH) & tri).astype(f32)
        k0 = keep_ref[0:1, b0:b0 + B]

        def self_cond(carry):
            return carry[1]

        def self_body(carry):
            k, _ = carry
            cnt = jnp.dot(k, sup_ref[...], preferred_element_type=f32)
            knew = k0 * (cnt < 0.5).astype(f32)
            return knew, jnp.any(knew != k)

        kb, _ = jax.lax.while_loop(self_cond, self_body,
                                   (k0, jnp.bool_(True)))
        keep_ref[0:1, b0:b0 + B] = kb

        def cross_body(rj, _):
            j0 = pl.multiple_of(rj * B, B)
            supm = (iou_tile(b0, j0) > _THRESH).astype(f32)
            cnt = jnp.dot(kb, supm, preferred_element_type=f32)
            keep_ref[0:1, pl.ds(j0, B)] = (keep_ref[0:1, pl.ds(j0, B)]
                                           * (cnt < 0.5).astype(f32))
            return 0

        if bi + 1 < nb:
            jax.lax.fori_loop(bi + 1, nb, cross_body, 0)

    out_ref[...] = out_ref[...] * keep_ref[...]


def kernel(pred_boxes, scores):
    N = pred_boxes.shape[0]
    B = _B
    P = ((N + B - 1) // B) * B
    pad = P - N
    boxes_p = jnp.pad(pred_boxes.astype(jnp.float32), ((0, pad), (0, 0)))
    scores_p = jnp.pad(scores.astype(jnp.float32), (0, pad),
                       constant_values=-1.0)
    vals8 = jnp.concatenate(
        [boxes_p.T, scores_p[None, :], jnp.zeros((3, P), jnp.float32)], axis=0)
    valsT = vals8.T
    out = pl.pallas_call(
        functools.partial(_nms_kernel, nb=P // B, B=B),
        out_shape=jax.ShapeDtypeStruct((8, P), jnp.float32),
        scratch_shapes=[
            pltpu.VMEM((P, 8), jnp.float32),
            pltpu.VMEM((P, 1), jnp.float32),
            pltpu.VMEM((1, P), jnp.float32),
            pltpu.VMEM((B, B), jnp.float32),
        ],
    )(vals8, valsT)
    return out[:5, :N].T


# speedup vs baseline: 52.6238x; 1.2095x over previous
import functools

import jax
import jax.numpy as jnp
from jax.experimental import pallas as pl
from jax.experimental.pallas import tpu as pltpu

_THRESH = 0.3
_B = 512


def _nms_kernel(vals_ref, valsT_ref, out_ref, srtT_ref,
                rankc_ref, keep_ref, sup_ref, *, nb, B):
    f32 = jnp.float32
    i32 = jnp.int32
    lane_b = jax.lax.broadcasted_iota(i32, (1, B), 1)
    sub_b = jax.lax.broadcasted_iota(i32, (B, 1), 0)

    ones_col = jnp.ones((B, 1), f32)
    for r in range(nb):
        b0 = r * B
        s_i = valsT_ref[b0:b0 + B, 4:5]
        i_ids = sub_b + b0

        def rank_body(c, cnt):
            c0 = pl.multiple_of(c * B, B)
            s_j = vals_ref[4:5, pl.ds(c0, B)]
            j_ids = lane_b + c * B
            beats = (s_j > s_i) | ((s_j == s_i) & (j_ids < i_ids))
            bf = jnp.where(beats, 1.0, 0.0)
            return cnt + jnp.dot(bf, ones_col, preferred_element_type=f32)

        cnt_c = jax.lax.fori_loop(0, nb, rank_body, jnp.zeros((B, 1), f32))
        rankc_ref[b0:b0 + B, :] = cnt_c

    for c in range(nb):
        c0 = c * B
        dest_f = (lane_b + c0).astype(f32)

        def perm_body(r, accT_c):
            r0 = pl.multiple_of(r * B, B)
            rc = rankc_ref[pl.ds(r0, B), :]
            oh2f = jnp.transpose(jnp.where(rc == dest_f, 1.0, 0.0))
            return accT_c + jnp.dot(oh2f, valsT_ref[pl.ds(r0, B), :],
                                    preferred_element_type=f32,
                                    precision=jax.lax.Precision.HIGHEST)

        accT_c = jax.lax.fori_loop(0, nb, perm_body, jnp.zeros((B, 8), f32))
        srtT_ref[c0:c0 + B, :] = accT_c
        out_ref[:, c0:c0 + B] = jnp.transpose(accT_c)

    keep_ref[...] = jnp.ones_like(keep_ref)
    tri = (jax.lax.broadcasted_iota(i32, (B, B), 1)
           > jax.lax.broadcasted_iota(i32, (B, B), 0))

    def iou_tile(i0, j0):
        x1c = srtT_ref[i0:i0 + B, 0:1]
        y1c = srtT_ref[i0:i0 + B, 1:2]
        x2c = srtT_ref[i0:i0 + B, 2:3]
        y2c = srtT_ref[i0:i0 + B, 3:4]
        a_col = (x2c - x1c + 1.0) * (y2c - y1c + 1.0)
        x1r = out_ref[0:1, pl.ds(j0, B)]
        y1r = out_ref[1:2, pl.ds(j0, B)]
        x2r = out_ref[2:3, pl.ds(j0, B)]
        y2r = out_ref[3:4, pl.ds(j0, B)]
        a_row = (x2r - x1r + 1.0) * (y2r - y1r + 1.0)
        xx1 = jnp.maximum(x1c, x1r)
        yy1 = jnp.maximum(y1c, y1r)
        xx2 = jnp.minimum(x2c, x2r)
        yy2 = jnp.minimum(y2c, y2r)
        iw = jnp.maximum(xx2 - xx1 + 1.0, 0.0)
        ih = jnp.maximum(yy2 - yy1 + 1.0, 0.0)
        inter = iw * ih
        union = (a_col + a_row) - inter
        return inter / union

    for bi in range(nb):
        b0 = bi * B
        sup_ref[...] = ((iou_tile(b0, b0) > _THRESH) & tri).astype(f32)
        k0 = keep_ref[0:1, b0:b0 + B]

        def self_cond(carry):
            return carry[1]

        def self_body(carry):
            k, _ = carry
            cnt = jnp.dot(k, sup_ref[...], preferred_element_type=f32)
            knew = k0 * (cnt < 0.5).astype(f32)
            return knew, jnp.any(knew != k)

        kb, _ = jax.lax.while_loop(self_cond, self_body,
                                   (k0, jnp.bool_(True)))
        keep_ref[0:1, b0:b0 + B] = kb

        def cross_body(rj, _):
            j0 = pl.multiple_of(rj * B, B)
            supm = (iou_tile(b0, j0) > _THRESH).astype(f32)
            cnt = jnp.dot(kb, supm, preferred_element_type=f32)
            keep_ref[0:1, pl.ds(j0, B)] = (keep_ref[0:1, pl.ds(j0, B)]
                                           * (cnt < 0.5).astype(f32))
            return 0

        if bi + 1 < nb:
            jax.lax.fori_loop(bi + 1, nb, cross_body, 0)

    out_ref[...] = out_ref[...] * keep_ref[...]


def kernel(pred_boxes, scores):
    N = pred_boxes.shape[0]
    B = _B
    P = ((N + B - 1) // B) * B
    pad = P - N
    boxes_p = jnp.pad(pred_boxes.astype(jnp.float32), ((0, pad), (0, 0)))
    scores_p = jnp.pad(scores.astype(jnp.float32), (0, pad),
                       constant_values=-1.0)
    vals8 = jnp.concatenate(
        [boxes_p.T, scores_p[None, :], jnp.zeros((3, P), jnp.float32)], axis=0)
    valsT = vals8.T
    out = pl.pallas_call(
        functools.partial(_nms_kernel, nb=P // B, B=B),
        out_shape=jax.ShapeDtypeStruct((8, P), jnp.float32),
        scratch_shapes=[
            pltpu.VMEM((P, 8), jnp.float32),
            pltpu.VMEM((P, 1), jnp.float32),
            pltpu.VMEM((1, P), jnp.float32),
            pltpu.VMEM((B, B), jnp.float32),
        ],
    )(vals8, valsT)
    return out[:5, :N].T


# speedup vs baseline: 85.1402x; 1.6179x over previous
import functools

import jax
import jax.numpy as jnp
from jax import lax
from jax.experimental import pallas as pl
from jax.experimental.pallas import tpu as pltpu
from jax.experimental.pallas import tpu_sc as plsc

_THRESH = 0.3
_B = 512


def _rank_kernel(vals_ref, valsT_ref, rank_ref, *, nb, B):
    f32 = jnp.float32
    i32 = jnp.int32
    lane_b = jax.lax.broadcasted_iota(i32, (1, B), 1)
    sub_b = jax.lax.broadcasted_iota(i32, (B, 1), 0)
    ones_col = jnp.ones((B, 1), f32)
    for r in range(nb):
        b0 = r * B
        s_i = valsT_ref[b0:b0 + B, 4:5]
        i_ids = sub_b + b0

        def rank_body(c, cnt):
            c0 = pl.multiple_of(c * B, B)
            s_j = vals_ref[4:5, pl.ds(c0, B)]
            j_ids = lane_b + c * B
            beats = (s_j > s_i) | ((s_j == s_i) & (j_ids < i_ids))
            bf = jnp.where(beats, 1.0, 0.0)
            return cnt + jnp.dot(bf, ones_col, preferred_element_type=f32)

        cnt_c = jax.lax.fori_loop(0, nb, rank_body, jnp.zeros((B, 1), f32))
        rank_ref[b0:b0 + B, :] = cnt_c.astype(i32)


def _nms_kernel(srt_ref, out_ref, keep_ref, sup_ref, *, nb, B):
    f32 = jnp.float32
    i32 = jnp.int32

    for c in range(nb):
        c0 = c * B
        out_ref[:, c0:c0 + B] = jnp.transpose(srt_ref[c0:c0 + B, 0:8])

    keep_ref[...] = jnp.ones_like(keep_ref)
    tri = (jax.lax.broadcasted_iota(i32, (B, B), 1)
           > jax.lax.broadcasted_iota(i32, (B, B), 0))

    def iou_tile(i0, j0):
        x1c = srt_ref[i0:i0 + B, 0:1]
        y1c = srt_ref[i0:i0 + B, 1:2]
        x2c = srt_ref[i0:i0 + B, 2:3]
        y2c = srt_ref[i0:i0 + B, 3:4]
        a_col = (x2c - x1c + 1.0) * (y2c - y1c + 1.0)
        x1r = out_ref[0:1, pl.ds(j0, B)]
        y1r = out_ref[1:2, pl.ds(j0, B)]
        x2r = out_ref[2:3, pl.ds(j0, B)]
        y2r = out_ref[3:4, pl.ds(j0, B)]
        a_row = (x2r - x1r + 1.0) * (y2r - y1r + 1.0)
        xx1 = jnp.maximum(x1c, x1r)
        yy1 = jnp.maximum(y1c, y1r)
        xx2 = jnp.minimum(x2c, x2r)
        yy2 = jnp.minimum(y2c, y2r)
        iw = jnp.maximum(xx2 - xx1 + 1.0, 0.0)
        ih = jnp.maximum(yy2 - yy1 + 1.0, 0.0)
        inter = iw * ih
        union = (a_col + a_row) - inter
        return inter / union

    for bi in range(nb):
        b0 = bi * B
        sup_ref[...] = ((iou_tile(b0, b0) > _THRESH) & tri).astype(f32)
        k0 = keep_ref[0:1, b0:b0 + B]

        def self_cond(carry):
            return carry[1]

        def self_body(carry):
            k, _ = carry
            cnt = jnp.dot(k, sup_ref[...], preferred_element_type=f32)
            knew = k0 * (cnt < 0.5).astype(f32)
            return knew, jnp.any(knew != k)

        kb, _ = jax.lax.while_loop(self_cond, self_body,
                                   (k0, jnp.bool_(True)))
        keep_ref[0:1, b0:b0 + B] = kb

        def cross_body(rj, _):
            j0 = pl.multiple_of(rj * B, B)
            supm = (iou_tile(b0, j0) > _THRESH).astype(f32)
            cnt = jnp.dot(kb, supm, preferred_element_type=f32)
            keep_ref[0:1, pl.ds(j0, B)] = (keep_ref[0:1, pl.ds(j0, B)]
                                           * (cnt < 0.5).astype(f32))
            return 0

        if bi + 1 < nb:
            jax.lax.fori_loop(bi + 1, nb, cross_body, 0)

    out_ref[...] = out_ref[...] * keep_ref[...]


def _sc_scatter(valsT128, rank1d, P):
    info = plsc.get_sparse_core_info()
    nw = info.num_cores * info.num_subcores
    b_per_w = P // nw
    mesh = plsc.VectorSubcoreMesh(core_axis_name="c", subcore_axis_name="s")

    @functools.partial(
        pl.kernel, mesh=mesh,
        out_type=jax.ShapeDtypeStruct((P, 128), jnp.float32),
        scratch_types=[
            pltpu.VMEM((b_per_w,), jnp.int32),
            pltpu.VMEM((b_per_w, 128), jnp.float32),
            pltpu.SemaphoreType.DMA,
        ],
    )
    def k(rows_hbm, idx_hbm, out_hbm, idx_v, rows_v, sem):
        wid = lax.axis_index("s") * info.num_cores + lax.axis_index("c")
        base = wid * b_per_w
        pltpu.sync_copy(idx_hbm.at[pl.ds(base, b_per_w)], idx_v)
        pltpu.sync_copy(rows_hbm.at[pl.ds(base, b_per_w)], rows_v)
        pltpu.async_copy(rows_v, out_hbm.at[idx_v], sem).wait()

    return k(valsT128, rank1d)


def kernel(pred_boxes, scores):
    N = pred_boxes.shape[0]
    B = _B
    P = ((N + B - 1) // B) * B
    pad = P - N
    boxes_p = jnp.pad(pred_boxes.astype(jnp.float32), ((0, pad), (0, 0)))
    scores_p = jnp.pad(scores.astype(jnp.float32), (0, pad),
                       constant_values=-1.0)
    vals8 = jnp.concatenate(
        [boxes_p.T, scores_p[None, :], jnp.zeros((3, P), jnp.float32)], axis=0)
    valsT = vals8.T
    rank = pl.pallas_call(
        functools.partial(_rank_kernel, nb=P // B, B=B),
        out_shape=jax.ShapeDtypeStruct((P, 1), jnp.int32),
    )(vals8, valsT)
    valsT128 = jnp.concatenate([valsT, jnp.zeros((P, 120), jnp.float32)], axis=1)
    srt = _sc_scatter(valsT128, rank[:, 0], P)
    out = pl.pallas_call(
        functools.partial(_nms_kernel, nb=P // B, B=B),
        out_shape=jax.ShapeDtypeStruct((8, P), jnp.float32),
        scratch_shapes=[
            pltpu.VMEM((1, P), jnp.float32),
            pltpu.VMEM((B, B), jnp.float32),
        ],
    )(srt)
    return out[:5, :N].T
